# baseline (speedup 1.0000x reference)
import jax
import jax.numpy as jnp
from jax import lax
from jax.experimental import pallas as pl
from jax.experimental.pallas import tpu as pltpu

_TOP_K = 8
_BLOCK = 2048
_KSPLIT = 2


def _router_body(x_ref, wt_ref, idx_ref, w_ref, p_ref, acc_ref):
    k = pl.program_id(1)
    part = jnp.dot(x_ref[...], wt_ref[...], preferred_element_type=jnp.float32)

    @pl.when(k == 0)
    def _init():
        acc_ref[...] = part

    @pl.when(k == _KSPLIT - 1)
    def _finish():
        logits = acc_ref[...] + part
        m = jnp.max(logits, axis=-1, keepdims=True)
        e = jnp.exp(logits - m)
        probs = e / jnp.sum(e, axis=-1, keepdims=True)
        p_ref[...] = probs
        ne = probs.shape[1]
        col = lax.broadcasted_iota(jnp.int32, probs.shape, 1)
        vals = probs
        idx_cols, w_cols = [], []
        for _ in range(_TOP_K):
            mj = jnp.max(vals, axis=-1, keepdims=True)
            amj = jnp.min(jnp.where(vals == mj, col, ne), axis=-1, keepdims=True)
            w_cols.append(mj)
            idx_cols.append(amj)
            vals = jnp.where(col == amj, -1.0, vals)
        idx_ref[...] = jnp.concatenate(idx_cols, axis=1)
        w_ref[...] = jnp.concatenate(w_cols, axis=1)


@jax.jit
def kernel(x, W):
    n, d = x.shape
    ne = W.shape[0]
    dk = d // _KSPLIT
    wt = W.T
    out = pl.pallas_call(
        _router_body,
        grid=(n // _BLOCK, _KSPLIT),
        in_specs=[
            pl.BlockSpec((_BLOCK, dk), lambda i, k: (i, k)),
            pl.BlockSpec((dk, ne), lambda i, k: (k, 0)),
        ],
        out_specs=[
            pl.BlockSpec((_BLOCK, _TOP_K), lambda i, k: (i, 0)),
            pl.BlockSpec((_BLOCK, _TOP_K), lambda i, k: (i, 0)),
            pl.BlockSpec((_BLOCK, ne), lambda i, k: (i, 0)),
        ],
        out_shape=[
            jax.ShapeDtypeStruct((n, _TOP_K), jnp.int32),
            jax.ShapeDtypeStruct((n, _TOP_K), jnp.float32),
            jax.ShapeDtypeStruct((n, ne), jnp.float32),
        ],
        scratch_shapes=[pltpu.VMEM((_BLOCK, ne), jnp.float32)],
        compiler_params=pltpu.CompilerParams(
            dimension_semantics=("parallel", "arbitrary")
        ),
    )(x, wt)
    indices, weights, probs = out
    return (indices, weights, probs)


# speedup vs baseline: 1.3445x; 1.3445x over previous
import jax
import jax.numpy as jnp
from jax import lax
from jax.experimental import pallas as pl
from jax.experimental.pallas import tpu as pltpu

_TOP_K = 8
_BLOCK = 1024
_CHUNK = 128


def _router_body(x_ref, wt_ref, idx_ref, w_ref, p_ref):
    logits = jnp.dot(x_ref[...], wt_ref[...], preferred_element_type=jnp.float32)
    ne = logits.shape[1]
    for r in range(0, _BLOCK, _CHUNK):
        lg = logits[r:r + _CHUNK, :]
        m = jnp.max(lg, axis=-1, keepdims=True)
        e = jnp.exp(lg - m)
        probs = e * (1.0 / jnp.sum(e, axis=-1, keepdims=True))
        p_ref[r:r + _CHUNK, :] = probs
        colf = lax.broadcasted_iota(jnp.int32, probs.shape, 1).astype(jnp.float32)
        vals = probs
        idx_cols, w_cols = [], []
        for _ in range(_TOP_K):
            mj = jnp.max(vals, axis=-1, keepdims=True)
            amj = jnp.min(jnp.where(vals == mj, colf, float(ne)),
                          axis=-1, keepdims=True)
            w_cols.append(mj)
            idx_cols.append(amj)
            vals = jnp.where(colf == amj, -1.0, vals)
        idx_ref[r:r + _CHUNK, :] = jnp.concatenate(idx_cols, axis=1).astype(jnp.int32)
        w_ref[r:r + _CHUNK, :] = jnp.concatenate(w_cols, axis=1)


@jax.jit
def kernel(x, W):
    n, d = x.shape
    ne = W.shape[0]
    wt = W.T
    out = pl.pallas_call(
        _router_body,
        grid=(n // _BLOCK,),
        in_specs=[
            pl.BlockSpec((_BLOCK, d), lambda i: (i, 0)),
            pl.BlockSpec((d, ne), lambda i: (0, 0)),
        ],
        out_specs=[
            pl.BlockSpec((_BLOCK, _TOP_K), lambda i: (i, 0)),
            pl.BlockSpec((_BLOCK, _TOP_K), lambda i: (i, 0)),
            pl.BlockSpec((_BLOCK, ne), lambda i: (i, 0)),
        ],
        out_shape=[
            jax.ShapeDtypeStruct((n, _TOP_K), jnp.int32),
            jax.ShapeDtypeStruct((n, _TOP_K), jnp.float32),
            jax.ShapeDtypeStruct((n, ne), jnp.float32),
        ],
        compiler_params=pltpu.CompilerParams(
            dimension_semantics=("parallel",)
        ),
    )(x, wt)
    indices, weights, probs = out
    return (indices, weights, probs)


# speedup vs baseline: 1.3500x; 1.0041x over previous
import jax
import jax.numpy as jnp
from jax import lax
from jax.experimental import pallas as pl
from jax.experimental.pallas import tpu as pltpu

_TOP_K = 8
_BLOCK = 1024
_CHUNK = 128


def _router_body(x_ref, wt_ref, idx_ref, w_ref, p_ref):
    logits = jnp.dot(x_ref[...], wt_ref[...], preferred_element_type=jnp.float32)
    ne = logits.shape[1]
    for r in range(0, _BLOCK, _CHUNK):
        lg = logits[r:r + _CHUNK, :]
        colf = lax.broadcasted_iota(jnp.int32, lg.shape, 1).astype(jnp.float32)
        vals = lg
        lg_cols, idx_cols = [], []
        for _ in range(_TOP_K):
            mj = jnp.max(vals, axis=-1, keepdims=True)
            amj = jnp.min(jnp.where(vals == mj, colf, float(ne)),
                          axis=-1, keepdims=True)
            lg_cols.append(mj)
            idx_cols.append(amj)
            vals = jnp.where(colf == amj, -jnp.inf, vals)
        m = lg_cols[0]
        e = jnp.exp(lg - m)
        rs = 1.0 / jnp.sum(e, axis=-1, keepdims=True)
        p_ref[r:r + _CHUNK, :] = e * rs
        sel = jnp.concatenate(lg_cols, axis=1)
        w_ref[r:r + _CHUNK, :] = jnp.exp(sel - m) * rs
        idx_ref[r:r + _CHUNK, :] = jnp.concatenate(idx_cols, axis=1).astype(jnp.int32)


@jax.jit
def kernel(x, W):
    n, d = x.shape
    ne = W.shape[0]
    wt = W.T
    out = pl.pallas_call(
        _router_body,
        grid=(n // _BLOCK,),
        in_specs=[
            pl.BlockSpec((_BLOCK, d), lambda i: (i, 0)),
            pl.BlockSpec((d, ne), lambda i: (0, 0)),
        ],
        out_specs=[
            pl.BlockSpec((_BLOCK, _TOP_K), lambda i: (i, 0)),
            pl.BlockSpec((_BLOCK, _TOP_K), lambda i: (i, 0)),
            pl.BlockSpec((_BLOCK, ne), lambda i: (i, 0)),
        ],
        out_shape=[
            jax.ShapeDtypeStruct((n, _TOP_K), jnp.int32),
            jax.ShapeDtypeStruct((n, _TOP_K), jnp.float32),
            jax.ShapeDtypeStruct((n, ne), jnp.float32),
        ],
        compiler_params=pltpu.CompilerParams(
            dimension_semantics=("parallel",)
        ),
    )(x, wt)
    indices, weights, probs = out
    return (indices, weights, probs)


# speedup vs baseline: 1.3934x; 1.0321x over previous
import jax
import jax.numpy as jnp
from jax import lax
from jax.experimental import pallas as pl
from jax.experimental.pallas import tpu as pltpu

_TOP_K = 8
_BLOCK = 1024
_CHUNK = 128


def _router_body(x_ref, wt_ref, idx_ref, w_ref, p_ref):
    wt = wt_ref[...]
    ne = wt.shape[1]
    for r in range(0, _BLOCK, _CHUNK):
        lg = jnp.dot(x_ref[r:r + _CHUNK, :], wt,
                     preferred_element_type=jnp.float32)
        colf = lax.broadcasted_iota(jnp.int32, lg.shape, 1).astype(jnp.float32)
        vals = lg
        lg_cols, idx_cols = [], []
        for _ in range(_TOP_K):
            mj = jnp.max(vals, axis=-1, keepdims=True)
            amj = jnp.min(jnp.where(vals == mj, colf, float(ne)),
                          axis=-1, keepdims=True)
            lg_cols.append(mj)
            idx_cols.append(amj)
            vals = jnp.where(colf == amj, -jnp.inf, vals)
        m = lg_cols[0]
        e = jnp.exp(lg - m)
        rs = 1.0 / jnp.sum(e, axis=-1, keepdims=True)
        p_ref[r:r + _CHUNK, :] = e * rs
        sel = jnp.concatenate(lg_cols, axis=1)
        w_ref[r:r + _CHUNK, :] = jnp.exp(sel - m) * rs
        idx_ref[r:r + _CHUNK, :] = jnp.concatenate(idx_cols, axis=1).astype(jnp.int32)


@jax.jit
def kernel(x, W):
    n, d = x.shape
    ne = W.shape[0]
    wt = W.T
    out = pl.pallas_call(
        _router_body,
        grid=(n // _BLOCK,),
        in_specs=[
            pl.BlockSpec((_BLOCK, d), lambda i: (i, 0)),
            pl.BlockSpec((d, ne), lambda i: (0, 0)),
        ],
        out_specs=[
            pl.BlockSpec((_BLOCK, _TOP_K), lambda i: (i, 0)),
            pl.BlockSpec((_BLOCK, _TOP_K), lambda i: (i, 0)),
            pl.BlockSpec((_BLOCK, ne), lambda i: (i, 0)),
        ],
        out_shape=[
            jax.ShapeDtypeStruct((n, _TOP_K), jnp.int32),
            jax.ShapeDtypeStruct((n, _TOP_K), jnp.float32),
            jax.ShapeDtypeStruct((n, ne), jnp.float32),
        ],
        compiler_params=pltpu.CompilerParams(
            dimension_semantics=("parallel",)
        ),
    )(x, wt)
    indices, weights, probs = out
    return (indices, weights, probs)


# speedup vs baseline: 1.3952x; 1.0013x over previous
import jax
import jax.numpy as jnp
from jax import lax
from jax.experimental import pallas as pl
from jax.experimental.pallas import tpu as pltpu

_TOP_K = 8
_BLOCK = 1024
_CHUNK = 256


def _router_body(x_ref, wt_ref, idx_ref, w_ref, p_ref):
    wt = wt_ref[...]
    ne = wt.shape[1]
    for r in range(0, _BLOCK, _CHUNK):
        lg = jnp.dot(x_ref[r:r + _CHUNK, :], wt,
                     preferred_element_type=jnp.float32)
        colf = lax.broadcasted_iota(jnp.int32, lg.shape, 1).astype(jnp.float32)
        vals = lg
        lg_cols, idx_cols = [], []
        for _ in range(_TOP_K):
            mj = jnp.max(vals, axis=-1, keepdims=True)
            amj = jnp.min(jnp.where(vals == mj, colf, float(ne)),
                          axis=-1, keepdims=True)
            lg_cols.append(mj)
            idx_cols.append(amj)
            vals = jnp.where(colf == amj, -jnp.inf, vals)
        m = lg_cols[0]
        e = jnp.exp(lg - m)
        rs = 1.0 / jnp.sum(e, axis=-1, keepdims=True)
        p_ref[r:r + _CHUNK, :] = e * rs
        sel = jnp.concatenate(lg_cols, axis=1)
        w_ref[r:r + _CHUNK, :] = jnp.exp(sel - m) * rs
        idx_ref[r:r + _CHUNK, :] = jnp.concatenate(idx_cols, axis=1).astype(jnp.int32)


@jax.jit
def kernel(x, W):
    n, d = x.shape
    ne = W.shape[0]
    wt = W.T
    out = pl.pallas_call(
        _router_body,
        grid=(n // _BLOCK,),
        in_specs=[
            pl.BlockSpec((_BLOCK, d), lambda i: (i, 0)),
            pl.BlockSpec((d, ne), lambda i: (0, 0)),
        ],
        out_specs=[
            pl.BlockSpec((_BLOCK, _TOP_K), lambda i: (i, 0)),
            pl.BlockSpec((_BLOCK, _TOP_K), lambda i: (i, 0)),
            pl.BlockSpec((_BLOCK, ne), lambda i: (i, 0)),
        ],
        out_shape=[
            jax.ShapeDtypeStruct((n, _TOP_K), jnp.int32),
            jax.ShapeDtypeStruct((n, _TOP_K), jnp.float32),
            jax.ShapeDtypeStruct((n, ne), jnp.float32),
        ],
        compiler_params=pltpu.CompilerParams(
            dimension_semantics=("parallel",)
        ),
    )(x, wt)
    indices, weights, probs = out
    return (indices, weights, probs)
